# Initial kernel scaffold; baseline (speedup 1.0000x reference)
import jax
import jax.numpy as jnp
from jax.experimental import pallas as pl


def kernel(x, edge_index, Wl1, Wr1, b1, g1, be1, Wl2, Wr2, b2, g2, be2, Wl3, Wr3, b3):
    raise NotImplementedError("write your pallas kernel here")


# speedup vs baseline: 3.6685x; 3.6685x over previous
import functools

import jax
import jax.numpy as jnp
from jax import lax
from jax.experimental import pallas as pl
from jax.experimental.pallas import tpu as pltpu
from jax.experimental.pallas import tpu_sc as plsc

N = 10000
NP = 10240
E = 320000
CHUNK = 128
NCHUNKS = 2560
EP = NCHUNKS * CHUNK
NCORES = 2
NSUB = 16
NWORKERS = NCORES * NSUB
CPW = NCHUNKS // NWORKERS
RPT = NP // NSUB
DH = 128
DOUT = 64


@functools.lru_cache(maxsize=None)
def _make_agg(D, with_counts=False):
    mesh = plsc.VectorSubcoreMesh(core_axis_name="c", subcore_axis_name="s")
    out_type = [jax.ShapeDtypeStruct((NCORES, NP, D), jnp.float32)]
    scratch = [
        pltpu.VMEM((CPW, CHUNK), jnp.int32),
        pltpu.VMEM((CPW, CHUNK), jnp.int32),
        pltpu.VMEM((CHUNK, D), jnp.float32),
        pltpu.VMEM((16, D), jnp.float32),
        pltpu.VMEM_SHARED((NP, D), jnp.float32),
        pltpu.SemaphoreType.DMA,
    ]
    if with_counts:
        out_type.append(jax.ShapeDtypeStruct((NCORES, NP), jnp.float32))
        scratch += [
            pltpu.VMEM((CHUNK,), jnp.float32),
            pltpu.VMEM((RPT,), jnp.float32),
            pltpu.VMEM_SHARED((NP,), jnp.float32),
        ]

    @functools.partial(
        pl.kernel,
        out_type=out_type,
        mesh=mesh,
        compiler_params=pltpu.CompilerParams(use_tc_tiling_on_sc=False),
        scratch_types=scratch,
    )
    def agg(h_hbm, src_hbm, dst_hbm, out_hbm, *rest):
        if with_counts:
            (cnt_hbm, src_v, dst_v, rows_v, zero_v, acc_sh, sem,
             ones_v, zrow_v, cacc_sh) = rest
        else:
            src_v, dst_v, rows_v, zero_v, acc_sh, sem = rest
        cid = lax.axis_index("c")
        sid = lax.axis_index("s")
        wid = cid * NSUB + sid

        z16 = jnp.zeros((16,), jnp.float32)
        for r in range(16):
            for k in range(D // 16):
                zero_v[r, pl.ds(k * 16, 16)] = z16
        base_row = sid * RPT

        def _zero_body(i, carry):
            pltpu.sync_copy(zero_v, acc_sh.at[pl.ds(base_row + i * 16, 16)])
            return carry

        lax.fori_loop(0, RPT // 16, _zero_body, 0)
        if with_counts:
            one16 = jnp.full((16,), 1.0, jnp.float32)
            for k in range(CHUNK // 16):
                ones_v[pl.ds(k * 16, 16)] = one16
            for k in range(RPT // 16):
                zrow_v[pl.ds(k * 16, 16)] = z16
            pltpu.sync_copy(zrow_v, cacc_sh.at[pl.ds(base_row, RPT)])
        plsc.subcore_barrier()

        pltpu.sync_copy(src_hbm.at[pl.ds(wid * CPW, CPW)], src_v)
        pltpu.sync_copy(dst_hbm.at[pl.ds(wid * CPW, CPW)], dst_v)

        def _edge_body(j, carry):
            pltpu.async_copy(h_hbm.at[src_v.at[j]], rows_v, sem).wait()
            pltpu.sync_copy(rows_v, acc_sh.at[dst_v.at[j]], add=True)
            if with_counts:
                pltpu.sync_copy(ones_v, cacc_sh.at[dst_v.at[j]], add=True)
            return carry

        lax.fori_loop(0, CPW, _edge_body, 0)
        plsc.subcore_barrier()

        pltpu.sync_copy(acc_sh.at[pl.ds(base_row, RPT)],
                        out_hbm.at[cid, pl.ds(base_row, RPT)])
        if with_counts:
            pltpu.sync_copy(cacc_sh.at[pl.ds(base_row, RPT)],
                            cnt_hbm.at[cid, pl.ds(base_row, RPT)])

    return agg


def _row_mask():
    return (lax.broadcasted_iota(jnp.int32, (NP, 1), 0) < N).astype(jnp.float32)


def _bn_relu(z, g, b):
    mask = _row_mask()
    m = jnp.sum(z * mask, axis=0) / N
    zc = z - m[None, :]
    v = jnp.sum(zc * zc * mask, axis=0) / N
    zn = zc / jnp.sqrt(v + 1e-5) * g[None, :] + b[None, :]
    return jnp.maximum(zn, 0.0)


def _tc1_body(p_ref, pc_ref, x_ref, wl_ref, wr_ref, b_ref, g_ref, be_ref,
              h_ref, cnt_ref):
    p = p_ref[...]
    s = p[0] + p[1]
    pc = pc_ref[...]
    cnt = pc[0] + pc[1]
    inv = 1.0 / jnp.maximum(cnt, 1.0)
    mean = s * inv[:, None]
    x = x_ref[...]
    z = (jnp.dot(mean, wl_ref[...].T, preferred_element_type=jnp.float32)
         + jnp.dot(x, wr_ref[...].T, preferred_element_type=jnp.float32)
         + b_ref[...][None, :])
    h_ref[...] = _bn_relu(z, g_ref[...], be_ref[...])
    cnt_ref[...] = cnt


def _tc2_body(p_ref, h1_ref, cnt_ref, wl_ref, wr_ref, b_ref, g_ref, be_ref,
              wl3_ref, h2_ref, p3_ref):
    p = p_ref[...]
    s = p[0] + p[1]
    inv = 1.0 / jnp.maximum(cnt_ref[...], 1.0)
    mean = s * inv[:, None]
    h1 = h1_ref[...]
    z = (jnp.dot(mean, wl_ref[...].T, preferred_element_type=jnp.float32)
         + jnp.dot(h1, wr_ref[...].T, preferred_element_type=jnp.float32)
         + b_ref[...][None, :])
    h2 = _bn_relu(z, g_ref[...], be_ref[...])
    h2_ref[...] = h2
    p3_ref[...] = jnp.dot(h2, wl3_ref[...].T,
                          preferred_element_type=jnp.float32)


def _tc3_body(p_ref, h2_ref, cnt_ref, wr_ref, b_ref, out_ref):
    p = p_ref[...]
    s = p[0] + p[1]
    inv = 1.0 / jnp.maximum(cnt_ref[...], 1.0)
    z = (s * inv[:, None]
         + jnp.dot(h2_ref[...], wr_ref[...].T,
                   preferred_element_type=jnp.float32)
         + b_ref[...][None, :])
    mx = jnp.max(z, axis=1, keepdims=True)
    lse = mx + jnp.log(jnp.sum(jnp.exp(z - mx), axis=1, keepdims=True))
    out_ref[...] = (z - lse)[:N]


_tc1 = pl.pallas_call(
    _tc1_body,
    out_shape=(jax.ShapeDtypeStruct((NP, DH), jnp.float32),
               jax.ShapeDtypeStruct((NP,), jnp.float32)),
)
_tc2 = pl.pallas_call(
    _tc2_body,
    out_shape=(jax.ShapeDtypeStruct((NP, DH), jnp.float32),
               jax.ShapeDtypeStruct((NP, DOUT), jnp.float32)),
)
_tc3 = pl.pallas_call(
    _tc3_body,
    out_shape=jax.ShapeDtypeStruct((N, DOUT), jnp.float32),
)


def kernel(x, edge_index, Wl1, Wr1, b1, g1, be1, Wl2, Wr2, b2, g2, be2,
           Wl3, Wr3, b3):
    pad = jnp.full((EP - E,), N, jnp.int32)
    src = jnp.concatenate([edge_index[0], pad]).reshape(NCHUNKS, CHUNK)
    dst = jnp.concatenate([edge_index[1], pad]).reshape(NCHUNKS, CHUNK)
    x_pad = jnp.zeros((NP, DH), jnp.float32).at[:N].set(x)

    p1, pc1 = _make_agg(DH, with_counts=True)(x_pad, src, dst)
    h1, cnt = _tc1(p1, pc1, x_pad, Wl1, Wr1, b1, g1, be1)
    (p2,) = _make_agg(DH)(h1, src, dst)
    h2, p3 = _tc2(p2, h1, cnt, Wl2, Wr2, b2, g2, be2, Wl3)
    (p3a,) = _make_agg(DOUT)(p3, src, dst)
    return _tc3(p3a, h2, cnt, Wr3, b3)


# speedup vs baseline: 4.3939x; 1.1977x over previous
import functools

import jax
import jax.numpy as jnp
from jax import lax
from jax.experimental import pallas as pl
from jax.experimental.pallas import tpu as pltpu
from jax.experimental.pallas import tpu_sc as plsc

N = 10000
NP = 10240
E = 320000
CHUNK = 128
NCHUNKS = 2560
EP = NCHUNKS * CHUNK
NCORES = 2
NSUB = 16
NWORKERS = NCORES * NSUB
CPW = NCHUNKS // NWORKERS
RPT = NP // NSUB
DH = 128
DOUT = 64
NBUF = 2


@functools.lru_cache(maxsize=None)
def _make_agg(D, with_counts=False):
    mesh = plsc.VectorSubcoreMesh(core_axis_name="c", subcore_axis_name="s")
    NSLOT = 2 * NBUF
    out_type = [jax.ShapeDtypeStruct((NCORES, NP, D), jnp.float32)]
    scratch = [
        pltpu.VMEM((NSLOT, 2, CHUNK), jnp.int32),
        pltpu.VMEM((16, D), jnp.float32),
        pltpu.VMEM_SHARED((NP, D), jnp.float32),
    ]
    scratch += [pltpu.VMEM((CHUNK, D), jnp.float32)] * NBUF
    scratch += [pltpu.SemaphoreType.DMA] * NSLOT
    scratch += [pltpu.SemaphoreType.DMA] * (2 * NBUF)
    if with_counts:
        out_type.append(jax.ShapeDtypeStruct((NCORES, NP), jnp.float32))
        scratch += [
            pltpu.VMEM((CHUNK,), jnp.float32),
            pltpu.VMEM((RPT,), jnp.float32),
            pltpu.VMEM_SHARED((NP,), jnp.float32),
            pltpu.SemaphoreType.DMA,
        ]

    @functools.partial(
        pl.kernel,
        out_type=out_type,
        mesh=mesh,
        compiler_params=pltpu.CompilerParams(use_tc_tiling_on_sc=False),
        scratch_types=scratch,
    )
    def agg(h_hbm, ep_hbm, out_hbm, *rest):
        if with_counts:
            cnt_hbm = rest[0]
            rest = rest[1:]
        idx_v, zero_v, acc_sh = rest[:3]
        rest = rest[3:]
        rows = rest[:NBUF]
        isem = rest[NBUF:NBUF + NSLOT]
        gsem = rest[NBUF + NSLOT:NBUF + NSLOT + NBUF]
        ssem = rest[NBUF + NSLOT + NBUF:NBUF + NSLOT + 2 * NBUF]
        if with_counts:
            ones_v, zrow_v, cacc_sh, csem = rest[NBUF + NSLOT + 2 * NBUF:]
        cid = lax.axis_index("c")
        sid = lax.axis_index("s")
        wid = cid * NSUB + sid
        cbase = wid * CPW

        z16 = jnp.zeros((16,), jnp.float32)
        for r in range(16):
            for k in range(D // 16):
                zero_v[r, pl.ds(k * 16, 16)] = z16
        base_row = sid * RPT

        def _zero_body(i, carry):
            pltpu.sync_copy(zero_v, acc_sh.at[pl.ds(base_row + i * 16, 16)])
            return carry

        lax.fori_loop(0, RPT // 16, _zero_body, 0)
        if with_counts:
            one16 = jnp.full((16,), 1.0, jnp.float32)
            for k in range(CHUNK // 16):
                ones_v[pl.ds(k * 16, 16)] = one16
            for k in range(RPT // 16):
                zrow_v[pl.ds(k * 16, 16)] = z16
            pltpu.sync_copy(zrow_v, cacc_sh.at[pl.ds(base_row, RPT)])
        plsc.subcore_barrier()

        for t in range(NSLOT):
            pltpu.async_copy(ep_hbm.at[cbase + t], idx_v.at[t], isem[t])
        for b in range(NBUF):
            pltpu.make_async_copy(ep_hbm.at[0], idx_v.at[b], isem[b]).wait()
            pltpu.async_copy(h_hbm.at[idx_v.at[b, 0]], rows[b], gsem[b])

        def _grp_body(g, carry):
            jb = g * NSLOT
            for p in range(NSLOT // NBUF):
                for b in range(NBUF):
                    t = p * NBUF + b
                    pltpu.make_async_copy(h_hbm.at[idx_v.at[t, 0]], rows[b],
                                          gsem[b]).wait()
                    pltpu.async_copy(rows[b], acc_sh.at[idx_v.at[t, 1]],
                                     ssem[b], add=True)
                    if with_counts:
                        pltpu.async_copy(ones_v, cacc_sh.at[idx_v.at[t, 1]],
                                         csem, add=True)
                for b in range(NBUF):
                    t = p * NBUF + b
                    tn = (t + NBUF) % NSLOT
                    pltpu.make_async_copy(rows[b], acc_sh.at[idx_v.at[t, 1]],
                                          ssem[b]).wait()
                    nl = jnp.minimum(cbase + jb + t + NSLOT,
                                     cbase + CPW - 1)
                    pltpu.async_copy(ep_hbm.at[nl], idx_v.at[t], isem[t])
                    pltpu.make_async_copy(ep_hbm.at[0], idx_v.at[tn],
                                          isem[tn]).wait()
                    pltpu.async_copy(h_hbm.at[idx_v.at[tn, 0]], rows[b],
                                     gsem[b])
            return carry

        lax.fori_loop(0, CPW // NSLOT, _grp_body, 0)
        for b in range(NBUF):
            pltpu.make_async_copy(h_hbm.at[idx_v.at[0, 0]], rows[b],
                                  gsem[b]).wait()
        for t in range(NBUF, NSLOT):
            pltpu.make_async_copy(ep_hbm.at[0], idx_v.at[t], isem[t]).wait()
        if with_counts:
            def _cdrain(j, carry):
                pltpu.make_async_copy(ones_v, cacc_sh.at[idx_v.at[0, 1]],
                                      csem).wait()
                return carry

            lax.fori_loop(0, CPW, _cdrain, 0)
        plsc.subcore_barrier()

        pltpu.sync_copy(acc_sh.at[pl.ds(base_row, RPT)],
                        out_hbm.at[cid, pl.ds(base_row, RPT)])
        if with_counts:
            pltpu.sync_copy(cacc_sh.at[pl.ds(base_row, RPT)],
                            cnt_hbm.at[cid, pl.ds(base_row, RPT)])

    return agg


def _row_mask():
    return (lax.broadcasted_iota(jnp.int32, (NP, 1), 0) < N).astype(jnp.float32)


def _bn_relu(z, g, b):
    mask = _row_mask()
    m = jnp.sum(z * mask, axis=0) / N
    zc = z - m[None, :]
    v = jnp.sum(zc * zc * mask, axis=0) / N
    zn = zc / jnp.sqrt(v + 1e-5) * g[None, :] + b[None, :]
    return jnp.maximum(zn, 0.0)


def _tc1_body(p_ref, pc_ref, x_ref, wl_ref, wr_ref, b_ref, g_ref, be_ref,
              h_ref, cnt_ref):
    p = p_ref[...]
    s = p[0] + p[1]
    pc = pc_ref[...]
    cnt = pc[0] + pc[1]
    inv = 1.0 / jnp.maximum(cnt, 1.0)
    mean = s * inv[:, None]
    x = x_ref[...]
    z = (jnp.dot(mean, wl_ref[...].T, preferred_element_type=jnp.float32)
         + jnp.dot(x, wr_ref[...].T, preferred_element_type=jnp.float32)
         + b_ref[...][None, :])
    h_ref[...] = _bn_relu(z, g_ref[...], be_ref[...])
    cnt_ref[...] = cnt


def _tc2_body(p_ref, h1_ref, cnt_ref, wl_ref, wr_ref, b_ref, g_ref, be_ref,
              wl3_ref, h2_ref, p3_ref):
    p = p_ref[...]
    s = p[0] + p[1]
    inv = 1.0 / jnp.maximum(cnt_ref[...], 1.0)
    mean = s * inv[:, None]
    h1 = h1_ref[...]
    z = (jnp.dot(mean, wl_ref[...].T, preferred_element_type=jnp.float32)
         + jnp.dot(h1, wr_ref[...].T, preferred_element_type=jnp.float32)
         + b_ref[...][None, :])
    h2 = _bn_relu(z, g_ref[...], be_ref[...])
    h2_ref[...] = h2
    p3_ref[...] = jnp.dot(h2, wl3_ref[...].T,
                          preferred_element_type=jnp.float32)


def _tc3_body(p_ref, h2_ref, cnt_ref, wr_ref, b_ref, out_ref):
    p = p_ref[...]
    s = p[0] + p[1]
    inv = 1.0 / jnp.maximum(cnt_ref[...], 1.0)
    z = (s * inv[:, None]
         + jnp.dot(h2_ref[...], wr_ref[...].T,
                   preferred_element_type=jnp.float32)
         + b_ref[...][None, :])
    mx = jnp.max(z, axis=1, keepdims=True)
    lse = mx + jnp.log(jnp.sum(jnp.exp(z - mx), axis=1, keepdims=True))
    out_ref[...] = (z - lse)[:N]


_tc1 = pl.pallas_call(
    _tc1_body,
    out_shape=(jax.ShapeDtypeStruct((NP, DH), jnp.float32),
               jax.ShapeDtypeStruct((NP,), jnp.float32)),
)
_tc2 = pl.pallas_call(
    _tc2_body,
    out_shape=(jax.ShapeDtypeStruct((NP, DH), jnp.float32),
               jax.ShapeDtypeStruct((NP, DOUT), jnp.float32)),
)
_tc3 = pl.pallas_call(
    _tc3_body,
    out_shape=jax.ShapeDtypeStruct((N, DOUT), jnp.float32),
)


def kernel(x, edge_index, Wl1, Wr1, b1, g1, be1, Wl2, Wr2, b2, g2, be2,
           Wl3, Wr3, b3):
    pad = jnp.full((2, EP - E), N, jnp.int32)
    ep = (jnp.concatenate([edge_index, pad], axis=1)
          .reshape(2, NCHUNKS, CHUNK).transpose(1, 0, 2))
    x_pad = jnp.zeros((NP, DH), jnp.float32).at[:N].set(x)

    p1, pc1 = _make_agg(DH, with_counts=True)(x_pad, ep)
    h1, cnt = _tc1(p1, pc1, x_pad, Wl1, Wr1, b1, g1, be1)
    (p2,) = _make_agg(DH)(h1, ep)
    h2, p3 = _tc2(p2, h1, cnt, Wl2, Wr2, b2, g2, be2, Wl3)
    (p3a,) = _make_agg(DOUT)(p3, ep)
    return _tc3(p3a, h2, cnt, Wr3, b3)
